# Initial kernel scaffold; baseline (speedup 1.0000x reference)
import jax
import jax.numpy as jnp
from jax.experimental import pallas as pl


def kernel(bp_scores, sequences, sequence_lengths):
    raise NotImplementedError("write your pallas kernel here")


# speedup vs baseline: 368.2549x; 368.2549x over previous
import functools

import numpy as np
import jax
import jax.numpy as jnp
from jax import lax
from jax.experimental import pallas as pl
from jax.experimental.pallas import tpu as pltpu
from jax.experimental.pallas import tpu_sc as plsc

B = 16
L = 1024
NW = 32
ROWS_PER_W = 512
RB = 8
NBLK = ROWS_PER_W // RB
NCH = L // 16
TSIZE = 304
ZIDX = 288


def _build_t2d() -> np.ndarray:
    V = np.zeros((4, 4), dtype=np.float32)
    for (x, y) in [(0, 3), (3, 0), (3, 2), (2, 1), (2, 3), (1, 2)]:
        V[x, y] = 1.0
    SE = 0.5 * V[:, :, None, None] * V[None, None, :, :]
    stacking = {
        (0, 3, 0, 3): 0.9, (0, 3, 2, 1): 1.1, (0, 3, 2, 3): 0.8,
        (2, 1, 0, 3): 1.1, (2, 1, 2, 1): 1.3, (2, 1, 2, 3): 1.0,
        (2, 3, 0, 3): 0.8, (2, 3, 2, 1): 1.0, (2, 3, 2, 3): 0.7,
    }
    for k, v in stacking.items():
        SE[k] = v
    r = np.arange(16)
    c = np.arange(16)
    M2 = SE[r[:, None] // 4, c[None, :] // 4, r[:, None] % 4, c[None, :] % 4]
    TAB = np.zeros(TSIZE, dtype=np.float32)
    for rr in range(16):
        TAB[17 * rr:17 * rr + 16] = M2[rr]
    return (TAB[:, None] + TAB[None, :]).reshape(-1)


_T2D = _build_t2d()


def _sc_update(bp_flat, ia, jv, t2d):
    mesh = plsc.VectorSubcoreMesh(
        core_axis_name="c", subcore_axis_name="s", num_cores=2, num_subcores=16
    )

    @functools.partial(
        pl.kernel,
        out_type=jax.ShapeDtypeStruct((B, L * L), jnp.float32),
        mesh=mesh,
        compiler_params=pltpu.CompilerParams(needs_layout_passes=False),
        scratch_types=[
            pltpu.VMEM((TSIZE * TSIZE,), jnp.float32),
            pltpu.VMEM((4 * L,), jnp.int32),
            pltpu.VMEM((4 * L,), jnp.int32),
            pltpu.VMEM((RB * L,), jnp.float32),
            pltpu.VMEM((RB * L,), jnp.float32),
        ],
    )
    def k(bp_hbm, ia_hbm, jv_hbm, t2d_hbm, out_hbm, t2d_v, ia_v, jv_v, in_v, out_v):
        cid = lax.axis_index("c")
        sid = lax.axis_index("s")
        wid = sid * 2 + cid
        batch = wid // 2
        half = wid % 2

        pltpu.sync_copy(t2d_hbm, t2d_v)
        pltpu.sync_copy(ia_hbm.at[batch], ia_v)
        pltpu.sync_copy(jv_hbm.at[batch], jv_v)

        iota = lax.iota(jnp.int32, 16)
        zrow = jnp.full((16,), ZIDX * TSIZE, jnp.int32)
        zcol = jnp.full((16,), ZIDX, jnp.int32)

        def block_body(blk, _):
            r0 = half * ROWS_PER_W + blk * RB
            pltpu.sync_copy(bp_hbm.at[batch, pl.ds(r0 * L, RB * L)], in_v)

            def row_body(r, _):
                i = r0 + r
                ia1 = plsc.load_gather(ia_v, [jnp.full((16,), i, jnp.int32)])
                ia2 = plsc.load_gather(ia_v, [jnp.full((16,), L + i, jnp.int32)])
                ia3 = plsc.load_gather(ia_v, [jnp.full((16,), 2 * L + i, jnp.int32)])
                ia4 = plsc.load_gather(ia_v, [jnp.full((16,), 3 * L + i, jnp.int32)])

                def chunk_body(ch, _):
                    j0 = ch * 16
                    d = iota + (j0 - i)
                    x_u = jnp.where(d > 3, ia1 + jv_v[pl.ds(j0, 16)], zrow)
                    y_u = jnp.where(d > 1, ia3 + jv_v[pl.ds(2 * L + j0, 16)], zcol)
                    x_l = jnp.where(d < -3, ia2 + jv_v[pl.ds(L + j0, 16)], zrow)
                    y_l = jnp.where(d < -1, ia4 + jv_v[pl.ds(3 * L + j0, 16)], zcol)
                    g_u = plsc.load_gather(t2d_v, [x_u + y_u])
                    g_l = plsc.load_gather(t2d_v, [x_l + y_l])
                    base = r * L + j0
                    out_v[pl.ds(base, 16)] = in_v[pl.ds(base, 16)] + g_u + g_l
                    return 0

                lax.fori_loop(0, NCH, chunk_body, 0)
                return 0

            lax.fori_loop(0, RB, row_body, 0)
            pltpu.sync_copy(out_v, out_hbm.at[batch, pl.ds(r0 * L, RB * L)])
            return 0

        lax.fori_loop(0, NBLK, block_body, 0)

    return k(bp_flat, ia, jv, t2d)


@jax.jit
def kernel(bp_scores, sequences, sequence_lengths):
    s = sequences.astype(jnp.int32)
    ln = sequence_lengths.astype(jnp.int32)[:, None]
    pos = jnp.arange(L, dtype=jnp.int32)[None, :]

    a = 4 * s + jnp.roll(s, -1, axis=1)
    bb = 4 * s + jnp.roll(s, 1, axis=1)
    ap = jnp.roll(a, 1, axis=1)
    bn = jnp.roll(bb, -1, axis=1)

    ia1 = jnp.where(pos < ln - 1, 17 * a, 272) * TSIZE
    jv1 = jnp.where(pos < ln, bb, 16) * TSIZE
    ia3 = jnp.where((pos >= 1) & (pos < ln), 17 * ap, 272)
    jv3 = jnp.where(pos < ln - 1, bn, 16)
    ia2 = jnp.where(pos < ln, bb, 16) * TSIZE
    jv2 = jnp.where(pos < ln - 1, 17 * a, 272) * TSIZE
    ia4 = jnp.where(pos < ln - 1, bn, 16)
    jv4 = jnp.where((pos >= 1) & (pos < ln), 17 * ap, 272)

    ia = jnp.stack([ia1, ia2, ia3, ia4], axis=1).reshape(B, 4 * L).astype(jnp.int32)
    jv = jnp.stack([jv1, jv2, jv3, jv4], axis=1).reshape(B, 4 * L).astype(jnp.int32)
    t2d = jnp.asarray(_T2D)

    out = _sc_update(bp_scores.reshape(B, L * L), ia, jv, t2d)
    return out.reshape(B, L, L)
